# Initial kernel scaffold; baseline (speedup 1.0000x reference)
import jax
import jax.numpy as jnp
from jax.experimental import pallas as pl


def kernel(inputs, router_logits, wi_0, wi_1, wo):
    raise NotImplementedError("write your pallas kernel here")


# speedup vs baseline: 6.9643x; 6.9643x over previous
import functools

import jax
import jax.numpy as jnp
from jax import lax
from jax.experimental import pallas as pl
from jax.experimental.pallas import tpu as pltpu
from jax.experimental.pallas import tpu_sc as plsc

E = 64
TK = 2
H = 768
I = 1024
T = 4096
A = T * TK
RBLK = 512
TM = 256
NW = 32
TPW = T // NW


def _routing_body(logits_ref, dest_ref, w_ref, offs_ref):
    logits = logits_ref[...]
    ids = lax.broadcasted_iota(jnp.int32, (T, E), 1)
    m1 = jnp.max(logits, axis=1, keepdims=True)
    a1 = jnp.min(jnp.where(logits == m1, ids, E), axis=1, keepdims=True)
    masked = jnp.where(ids == a1, -jnp.inf, logits)
    m2 = jnp.max(masked, axis=1, keepdims=True)
    a2 = jnp.min(jnp.where(masked == m2, ids, E), axis=1, keepdims=True)

    l1 = m1.astype(jnp.bfloat16).astype(jnp.float32)
    l2 = m2.astype(jnp.bfloat16).astype(jnp.float32)
    mx = jnp.maximum(l1, l2)
    e1 = jnp.exp(l1 - mx)
    e2 = jnp.exp(l2 - mx)
    s = e1 + e2
    w1 = e1 / s
    w2 = e2 / s

    oh0 = (ids == a1).astype(jnp.float32)
    oh1 = (ids == a2).astype(jnp.float32)
    both = oh0 + oh1

    r = lax.broadcasted_iota(jnp.int32, (RBLK, RBLK), 0)
    c = lax.broadcasted_iota(jnp.int32, (RBLK, RBLK), 1)
    tri = (c <= r).astype(jnp.float32)
    run = jnp.zeros((1, E), jnp.float32)
    cex_blocks = []
    for b in range(T // RBLK):
        blk = both[b * RBLK:(b + 1) * RBLK, :]
        inc = jnp.dot(tri, blk, preferred_element_type=jnp.float32)
        cex_blocks.append(inc - blk + run)
        run = run + inc[RBLK - 1:RBLK, :]
    cex = jnp.concatenate(cex_blocks, axis=0)
    hist = run

    er = lax.broadcasted_iota(jnp.int32, (E, E), 0)
    ec = lax.broadcasted_iota(jnp.int32, (E, E), 1)
    strict = (er < ec).astype(jnp.float32)
    offs = jnp.dot(hist, strict, preferred_element_type=jnp.float32)

    pos = cex + offs
    d0 = jnp.sum(oh0 * pos, axis=1, keepdims=True)
    d1 = jnp.sum(oh1 * pos, axis=1, keepdims=True)
    dest_ref[...] = jnp.concatenate([d0, d1], axis=1).astype(jnp.int32)
    w_ref[...] = jnp.concatenate([w1, w2], axis=1).astype(jnp.bfloat16)
    pad = jnp.zeros((6, E), jnp.float32)
    offs_ref[...] = jnp.concatenate([offs, hist, pad], axis=0).astype(jnp.int32)


def _routing(router_logits, interpret=False):
    return pl.pallas_call(
        _routing_body,
        out_shape=[
            jax.ShapeDtypeStruct((T, TK), jnp.int32),
            jax.ShapeDtypeStruct((T, TK), jnp.bfloat16),
            jax.ShapeDtypeStruct((8, E), jnp.int32),
        ],
        interpret=interpret,
    )(router_logits)


HW = H // 2


def _to_i32(a):
    return lax.bitcast_convert_type(
        a.reshape(a.shape[0], a.shape[1] // 2, 2), jnp.int32)


def _to_bf16(a):
    return lax.bitcast_convert_type(a, jnp.bfloat16).reshape(
        a.shape[0], a.shape[1] * 2)


def _sc_mesh():
    return plsc.VectorSubcoreMesh(core_axis_name="c", subcore_axis_name="s")


def _dispatch(x_bf, dest_sm):
    @functools.partial(
        pl.kernel,
        out_type=jax.ShapeDtypeStruct((A, HW), jnp.int32),
        mesh=_sc_mesh(),
        scratch_types=[
            pltpu.VMEM((TPW,), jnp.int32),
            pltpu.VMEM((TPW, HW), jnp.int32),
            pltpu.SemaphoreType.DMA,
        ],
    )
    def k(x_hbm, dest_hbm, out_hbm, idx_v, rows_v, sem):
        wid = lax.axis_index("s") * 2 + lax.axis_index("c")
        base = wid * TPW
        pltpu.sync_copy(x_hbm.at[pl.ds(base, TPW)], rows_v)
        for s in range(TK):
            pltpu.sync_copy(dest_hbm.at[s, pl.ds(base, TPW)], idx_v)
            pltpu.async_copy(rows_v, out_hbm.at[idx_v], sem).wait()

    return k(x_bf, dest_sm)


def _gmm_body(offs_ref, x_ref, wi0_ref, wi1_ref, wo_ref, out_ref):
    e = pl.program_id(0)
    start = offs_ref[e]
    end = offs_ref[e + 1]
    astart = (start // 8) * 8
    nchunks = (end - astart + TM - 1) // TM

    def body(i, carry):
        base = jnp.minimum(astart + i * TM, A - TM)
        base = pl.multiple_of(base, 8)
        xs = x_ref[pl.ds(base, TM), :]
        a = jnp.dot(xs, wi0_ref[0], preferred_element_type=jnp.float32)
        b = jnp.dot(xs, wi1_ref[0], preferred_element_type=jnp.float32)
        a = a.astype(jnp.bfloat16).astype(jnp.float32)
        h = (a * jax.nn.sigmoid(a)).astype(jnp.bfloat16) * b.astype(jnp.bfloat16)
        y = jnp.dot(h, wo_ref[0], preferred_element_type=jnp.float32)
        y = y.astype(jnp.bfloat16)
        rows = base + lax.broadcasted_iota(jnp.int32, (TM, 1), 0)
        mask = (rows >= start) & (rows < end)
        cur = out_ref[pl.ds(base, TM), :]
        out_ref[pl.ds(base, TM), :] = jnp.where(mask, y, cur)
        return carry

    lax.fori_loop(0, nchunks, body, 0)


def _gmm(offs65, x_sorted, wi_0, wi_1, wo, interpret=False):
    grid_spec = pltpu.PrefetchScalarGridSpec(
        num_scalar_prefetch=1,
        grid=(E,),
        in_specs=[
            pl.BlockSpec((A, H), lambda e, offs: (0, 0)),
            pl.BlockSpec((1, H, I), lambda e, offs: (e, 0, 0)),
            pl.BlockSpec((1, H, I), lambda e, offs: (e, 0, 0)),
            pl.BlockSpec((1, I, H), lambda e, offs: (e, 0, 0)),
        ],
        out_specs=pl.BlockSpec((A, H), lambda e, offs: (0, 0)),
    )
    return pl.pallas_call(
        _gmm_body,
        grid_spec=grid_spec,
        out_shape=jax.ShapeDtypeStruct((A, H), jnp.bfloat16),
        compiler_params=pltpu.CompilerParams(
            dimension_semantics=("arbitrary",)),
        interpret=interpret,
    )(offs65, x_sorted, wi_0, wi_1, wo)


def _gather_outputs(y, dest_sm):
    @functools.partial(
        pl.kernel,
        out_type=[
            jax.ShapeDtypeStruct((T, HW), jnp.int32),
            jax.ShapeDtypeStruct((T, HW), jnp.int32),
        ],
        mesh=_sc_mesh(),
        scratch_types=[
            pltpu.VMEM((TPW,), jnp.int32),
            pltpu.VMEM((TPW, HW), jnp.int32),
            pltpu.SemaphoreType.DMA,
        ],
    )
    def k(y_hbm, dest_hbm, y0_hbm, y1_hbm, idx_v, rows_v, sem):
        wid = lax.axis_index("s") * 2 + lax.axis_index("c")
        base = wid * TPW
        for s, out in ((0, y0_hbm), (1, y1_hbm)):
            pltpu.sync_copy(dest_hbm.at[s, pl.ds(base, TPW)], idx_v)
            pltpu.async_copy(y_hbm.at[idx_v], rows_v, sem).wait()
            pltpu.sync_copy(rows_v, out.at[pl.ds(base, TPW)])

    return k(y, dest_sm)


def _combine_body(w_ref, y0_ref, y1_ref, out_ref):
    w = w_ref[...]
    out = y0_ref[...] * w[:, 0:1] + y1_ref[...] * w[:, 1:2]
    out_ref[...] = out.astype(jnp.float32)


def _combine(w_cols, y0, y1, interpret=False):
    nblk = 8
    blk = T // nblk
    return pl.pallas_call(
        _combine_body,
        grid=(nblk,),
        in_specs=[
            pl.BlockSpec((blk, TK), lambda i: (i, 0)),
            pl.BlockSpec((blk, H), lambda i: (i, 0)),
            pl.BlockSpec((blk, H), lambda i: (i, 0)),
        ],
        out_specs=pl.BlockSpec((blk, H), lambda i: (i, 0)),
        out_shape=jax.ShapeDtypeStruct((T, H), jnp.float32),
        interpret=interpret,
    )(w_cols, y0, y1)


def kernel(inputs, router_logits, wi_0, wi_1, wo):
    x = inputs.astype(jnp.bfloat16)
    dest_cols, w_cols, offs_rows = _routing(router_logits)
    dest_sm = dest_cols.T
    offs65 = jnp.concatenate(
        [offs_rows[0], jnp.array([A], jnp.int32)])
    x_sorted = _to_bf16(_dispatch(_to_i32(x), dest_sm))
    y = _gmm(offs65, x_sorted, wi_0, wi_1, wo)
    y0, y1 = _gather_outputs(_to_i32(y), dest_sm)
    return _combine(w_cols, _to_bf16(y0), _to_bf16(y1))


# speedup vs baseline: 17.7363x; 2.5468x over previous
import functools

import jax
import jax.numpy as jnp
from jax import lax
from jax.experimental import pallas as pl
from jax.experimental.pallas import tpu as pltpu
from jax.experimental.pallas import tpu_sc as plsc

E = 64
TK = 2
H = 768
I = 1024
T = 4096
A = T * TK
RBLK = 512
TM = 256
NW = 32
TPW = T // NW
CHK = 64
LN = 16


def _routing_body(logits_ref, dest_ref, w_ref, offs_ref):
    logits = logits_ref[...]
    ids = lax.broadcasted_iota(jnp.int32, (T, E), 1)
    m1 = jnp.max(logits, axis=1, keepdims=True)
    a1 = jnp.min(jnp.where(logits == m1, ids, E), axis=1, keepdims=True)
    masked = jnp.where(ids == a1, -jnp.inf, logits)
    m2 = jnp.max(masked, axis=1, keepdims=True)
    a2 = jnp.min(jnp.where(masked == m2, ids, E), axis=1, keepdims=True)

    l1 = m1.astype(jnp.bfloat16).astype(jnp.float32)
    l2 = m2.astype(jnp.bfloat16).astype(jnp.float32)
    mx = jnp.maximum(l1, l2)
    e1 = jnp.exp(l1 - mx)
    e2 = jnp.exp(l2 - mx)
    s = e1 + e2
    w1 = (e1 / s).astype(jnp.bfloat16).astype(jnp.float32)
    w2 = (e2 / s).astype(jnp.bfloat16).astype(jnp.float32)

    oh0 = (ids == a1).astype(jnp.float32)
    oh1 = (ids == a2).astype(jnp.float32)
    both = oh0 + oh1

    r = lax.broadcasted_iota(jnp.int32, (RBLK, RBLK), 0)
    c = lax.broadcasted_iota(jnp.int32, (RBLK, RBLK), 1)
    tri = (c <= r).astype(jnp.float32)
    run = jnp.zeros((1, E), jnp.float32)
    cex_blocks = []
    for b in range(T // RBLK):
        blk = both[b * RBLK:(b + 1) * RBLK, :]
        inc = jnp.dot(tri, blk, preferred_element_type=jnp.float32)
        cex_blocks.append(inc - blk + run)
        run = run + inc[RBLK - 1:RBLK, :]
    cex = jnp.concatenate(cex_blocks, axis=0)
    hist = run

    er = lax.broadcasted_iota(jnp.int32, (E, E), 0)
    ec = lax.broadcasted_iota(jnp.int32, (E, E), 1)
    strict = (er < ec).astype(jnp.float32)
    offs = jnp.dot(hist, strict, preferred_element_type=jnp.float32)

    pos = cex + offs
    d0 = jnp.sum(oh0 * pos, axis=1, keepdims=True)
    d1 = jnp.sum(oh1 * pos, axis=1, keepdims=True)
    dest_ref[...] = jnp.concatenate([d0, d1], axis=1).astype(jnp.int32)
    w_ref[...] = jnp.concatenate([w1, w2], axis=1)
    pad = jnp.zeros((6, E), jnp.float32)
    offs_ref[...] = jnp.concatenate([offs, hist, pad], axis=0).astype(jnp.int32)


def _routing(router_logits, interpret=False):
    return pl.pallas_call(
        _routing_body,
        out_shape=[
            jax.ShapeDtypeStruct((T, TK), jnp.int32),
            jax.ShapeDtypeStruct((T, TK), jnp.float32),
            jax.ShapeDtypeStruct((8, E), jnp.int32),
        ],
        interpret=interpret,
    )(router_logits)


def _sc_mesh():
    return plsc.VectorSubcoreMesh(core_axis_name="c", subcore_axis_name="s")


def _dispatch(x_f32, dest_sm):
    @functools.partial(
        pl.kernel,
        out_type=jax.ShapeDtypeStruct((A, H), jnp.float32),
        mesh=_sc_mesh(),
        scratch_types=[
            pltpu.VMEM((TPW,), jnp.int32),
            pltpu.VMEM((TPW, H), jnp.float32),
            pltpu.SemaphoreType.DMA,
        ],
    )
    def k(x_hbm, dest_hbm, out_hbm, idx_v, rows_v, sem):
        wid = lax.axis_index("s") * 2 + lax.axis_index("c")
        base = wid * TPW
        pltpu.sync_copy(x_hbm.at[pl.ds(base, TPW)], rows_v)
        for s in range(TK):
            pltpu.sync_copy(dest_hbm.at[s, pl.ds(base, TPW)], idx_v)
            pltpu.async_copy(rows_v, out_hbm.at[idx_v], sem).wait()

    return k(x_f32, dest_sm)


def _repack_body(xf_ref, xb_ref):
    xb_ref[...] = xf_ref[...].astype(jnp.bfloat16)


def _repack(x_sorted_f32, interpret=False):
    nblk = 8
    blk = A // nblk
    return pl.pallas_call(
        _repack_body,
        grid=(nblk,),
        in_specs=[pl.BlockSpec((blk, H), lambda i: (i, 0))],
        out_specs=pl.BlockSpec((blk, H), lambda i: (i, 0)),
        out_shape=jax.ShapeDtypeStruct((A, H), jnp.bfloat16),
        interpret=interpret,
    )(x_sorted_f32)


def _gmm_body(offs_ref, x_ref, wi0_ref, wi1_ref, wo_ref, out_ref):
    e = pl.program_id(0)
    start = offs_ref[e]
    end = offs_ref[e + 1]
    astart = (start // 8) * 8
    nchunks = (end - astart + TM - 1) // TM

    def body(i, carry):
        base = jnp.minimum(astart + i * TM, A - TM)
        base = pl.multiple_of(base, 8)
        xs = x_ref[pl.ds(base, TM), :]
        a = jnp.dot(xs, wi0_ref[0], preferred_element_type=jnp.float32)
        b = jnp.dot(xs, wi1_ref[0], preferred_element_type=jnp.float32)
        a = a.astype(jnp.bfloat16).astype(jnp.float32)
        h = (a * jax.nn.sigmoid(a)).astype(jnp.bfloat16) * b.astype(jnp.bfloat16)
        y = jnp.dot(h, wo_ref[0], preferred_element_type=jnp.float32)
        y = y.astype(jnp.bfloat16).astype(jnp.float32)
        rows = base + lax.broadcasted_iota(jnp.int32, (TM, 1), 0)
        mask = (rows >= start) & (rows < end)
        cur = out_ref[pl.ds(base, TM), :]
        out_ref[pl.ds(base, TM), :] = jnp.where(mask, y, cur)
        return carry

    lax.fori_loop(0, nchunks, body, 0)


def _gmm(offs65, x_sorted, wi_0, wi_1, wo, interpret=False):
    grid_spec = pltpu.PrefetchScalarGridSpec(
        num_scalar_prefetch=1,
        grid=(E,),
        in_specs=[
            pl.BlockSpec((A, H), lambda e, offs: (0, 0)),
            pl.BlockSpec((1, H, I), lambda e, offs: (e, 0, 0)),
            pl.BlockSpec((1, H, I), lambda e, offs: (e, 0, 0)),
            pl.BlockSpec((1, I, H), lambda e, offs: (e, 0, 0)),
        ],
        out_specs=pl.BlockSpec((A, H), lambda e, offs: (0, 0)),
    )
    return pl.pallas_call(
        _gmm_body,
        grid_spec=grid_spec,
        out_shape=jax.ShapeDtypeStruct((A, H), jnp.float32),
        compiler_params=pltpu.CompilerParams(
            dimension_semantics=("arbitrary",)),
        interpret=interpret,
    )(offs65, x_sorted, wi_0, wi_1, wo)


def _gather_outputs(y, dest_sm):
    @functools.partial(
        pl.kernel,
        out_type=[
            jax.ShapeDtypeStruct((T, H), jnp.float32),
            jax.ShapeDtypeStruct((T, H), jnp.float32),
        ],
        mesh=_sc_mesh(),
        scratch_types=[
            pltpu.VMEM((CHK,), jnp.int32),
            pltpu.VMEM((CHK,), jnp.int32),
            pltpu.VMEM((CHK, H), jnp.float32),
            pltpu.VMEM((CHK, H), jnp.float32),
            pltpu.SemaphoreType.DMA,
        ],
    )
    def k(y_hbm, dest_hbm, y0_hbm, y1_hbm, idx0_v, idx1_v, r0_v, r1_v, sem):
        wid = lax.axis_index("s") * 2 + lax.axis_index("c")
        for half in range(TPW // CHK):
            tok = wid * TPW + half * CHK
            pltpu.sync_copy(dest_hbm.at[0, pl.ds(tok, CHK)], idx0_v)
            pltpu.sync_copy(dest_hbm.at[1, pl.ds(tok, CHK)], idx1_v)
            cp0 = pltpu.async_copy(y_hbm.at[idx0_v], r0_v, sem)
            cp1 = pltpu.async_copy(y_hbm.at[idx1_v], r1_v, sem)
            cp0.wait()
            cp1.wait()
            pltpu.sync_copy(r0_v, y0_hbm.at[pl.ds(tok, CHK)])
            pltpu.sync_copy(r1_v, y1_hbm.at[pl.ds(tok, CHK)])

    return k(y, dest_sm)


def _combine_body(w_ref, y0_ref, y1_ref, out_ref):
    i = pl.program_id(0)
    n = y0_ref.shape[0]
    w = w_ref[pl.ds(i * n, n), :]
    out_ref[...] = y0_ref[...] * w[:, 0:1] + y1_ref[...] * w[:, 1:2]


def _combine(w_cols, y0, y1, interpret=False):
    nblk = 8
    blk = T // nblk
    return pl.pallas_call(
        _combine_body,
        grid=(nblk,),
        in_specs=[
            pl.BlockSpec((T, TK), lambda i: (0, 0)),
            pl.BlockSpec((blk, H), lambda i: (i, 0)),
            pl.BlockSpec((blk, H), lambda i: (i, 0)),
        ],
        out_specs=pl.BlockSpec((blk, H), lambda i: (i, 0)),
        out_shape=jax.ShapeDtypeStruct((T, H), jnp.float32),
        interpret=interpret,
    )(w_cols, y0, y1)


def kernel(inputs, router_logits, wi_0, wi_1, wo):
    dest_cols, w_cols, offs_rows = _routing(router_logits)
    dest_sm = dest_cols.T
    offs65 = jnp.concatenate(
        [offs_rows[0], jnp.array([A], jnp.int32)])
    x_sorted = _repack(_dispatch(inputs, dest_sm))
    y = _gmm(offs65, x_sorted, wi_0, wi_1, wo)
    y0, y1 = _gather_outputs(y, dest_sm)
    return _combine(w_cols, y0, y1)


# speedup vs baseline: 18.6290x; 1.0503x over previous
import functools

import jax
import jax.numpy as jnp
from jax import lax
from jax.experimental import pallas as pl
from jax.experimental.pallas import tpu as pltpu
from jax.experimental.pallas import tpu_sc as plsc

E = 64
TK = 2
H = 768
I = 1024
T = 4096
A = T * TK
RBLK = 512
TM = 256
NW = 32
TPW = T // NW
CHK = 64
LN = 16


def _routing_body(logits_ref, dest_ref, w_ref, offs_ref):
    logits = logits_ref[...]
    ids = lax.broadcasted_iota(jnp.int32, (T, E), 1)
    m1 = jnp.max(logits, axis=1, keepdims=True)
    a1 = jnp.min(jnp.where(logits == m1, ids, E), axis=1, keepdims=True)
    masked = jnp.where(ids == a1, -jnp.inf, logits)
    m2 = jnp.max(masked, axis=1, keepdims=True)
    a2 = jnp.min(jnp.where(masked == m2, ids, E), axis=1, keepdims=True)

    l1 = m1.astype(jnp.bfloat16).astype(jnp.float32)
    l2 = m2.astype(jnp.bfloat16).astype(jnp.float32)
    mx = jnp.maximum(l1, l2)
    e1 = jnp.exp(l1 - mx)
    e2 = jnp.exp(l2 - mx)
    s = e1 + e2
    w1 = (e1 / s).astype(jnp.bfloat16).astype(jnp.float32)
    w2 = (e2 / s).astype(jnp.bfloat16).astype(jnp.float32)

    oh0 = (ids == a1).astype(jnp.float32)
    oh1 = (ids == a2).astype(jnp.float32)
    both = oh0 + oh1

    r = lax.broadcasted_iota(jnp.int32, (RBLK, RBLK), 0)
    c = lax.broadcasted_iota(jnp.int32, (RBLK, RBLK), 1)
    tri = (c <= r).astype(jnp.float32)
    run = jnp.zeros((1, E), jnp.float32)
    cex_blocks = []
    for b in range(T // RBLK):
        blk = both[b * RBLK:(b + 1) * RBLK, :]
        inc = jnp.dot(tri, blk, preferred_element_type=jnp.float32)
        cex_blocks.append(inc - blk + run)
        run = run + inc[RBLK - 1:RBLK, :]
    cex = jnp.concatenate(cex_blocks, axis=0)
    hist = run

    er = lax.broadcasted_iota(jnp.int32, (E, E), 0)
    ec = lax.broadcasted_iota(jnp.int32, (E, E), 1)
    strict = (er < ec).astype(jnp.float32)
    offs = jnp.dot(hist, strict, preferred_element_type=jnp.float32)

    pos = cex + offs
    d0 = jnp.sum(oh0 * pos, axis=1, keepdims=True)
    d1 = jnp.sum(oh1 * pos, axis=1, keepdims=True)
    dest_ref[...] = jnp.concatenate([d0, d1], axis=1).astype(jnp.int32)
    w_ref[...] = jnp.concatenate([w1, w2], axis=1)
    pad = jnp.zeros((6, E), jnp.float32)
    offs_ref[...] = jnp.concatenate([offs, hist, pad], axis=0).astype(jnp.int32)


def _routing(router_logits, interpret=False):
    return pl.pallas_call(
        _routing_body,
        out_shape=[
            jax.ShapeDtypeStruct((T, TK), jnp.int32),
            jax.ShapeDtypeStruct((T, TK), jnp.float32),
            jax.ShapeDtypeStruct((8, E), jnp.int32),
        ],
        interpret=interpret,
    )(router_logits)


def _sc_mesh():
    return plsc.VectorSubcoreMesh(core_axis_name="c", subcore_axis_name="s")


def _dispatch(x_f32, dest_sm):
    @functools.partial(
        pl.kernel,
        out_type=jax.ShapeDtypeStruct((A, H), jnp.float32),
        mesh=_sc_mesh(),
        scratch_types=[
            pltpu.VMEM((TPW,), jnp.int32),
            pltpu.VMEM((TPW, H), jnp.float32),
            pltpu.SemaphoreType.DMA,
        ],
    )
    def k(x_hbm, dest_hbm, out_hbm, idx_v, rows_v, sem):
        wid = lax.axis_index("s") * 2 + lax.axis_index("c")
        base = wid * TPW
        pltpu.sync_copy(x_hbm.at[pl.ds(base, TPW)], rows_v)
        for s in range(TK):
            pltpu.sync_copy(dest_hbm.at[s, pl.ds(base, TPW)], idx_v)
            pltpu.async_copy(rows_v, out_hbm.at[idx_v], sem).wait()

    return k(x_f32, dest_sm)


def _repack_body(xf_ref, xb_ref):
    xb_ref[...] = xf_ref[...].astype(jnp.bfloat16)


def _repack(x_sorted_f32, interpret=False):
    nblk = 8
    blk = A // nblk
    return pl.pallas_call(
        _repack_body,
        grid=(nblk,),
        in_specs=[pl.BlockSpec((blk, H), lambda i: (i, 0))],
        out_specs=pl.BlockSpec((blk, H), lambda i: (i, 0)),
        out_shape=jax.ShapeDtypeStruct((A, H), jnp.bfloat16),
        interpret=interpret,
    )(x_sorted_f32)


def _gmm_body(offs_ref, x_ref, wi0_ref, wi1_ref, wo_ref, out_ref):
    e = pl.program_id(0)
    start = offs_ref[e]
    end = offs_ref[e + 1]
    astart = (start // 8) * 8
    nchunks = (end - astart + TM - 1) // TM

    def body(i, carry):
        base = jnp.minimum(astart + i * TM, A - TM)
        base = pl.multiple_of(base, 8)
        xs = x_ref[pl.ds(base, TM), :].astype(jnp.bfloat16)
        a = jnp.dot(xs, wi0_ref[0], preferred_element_type=jnp.float32)
        b = jnp.dot(xs, wi1_ref[0], preferred_element_type=jnp.float32)
        a = a.astype(jnp.bfloat16).astype(jnp.float32)
        h = (a * jax.nn.sigmoid(a)).astype(jnp.bfloat16) * b.astype(jnp.bfloat16)
        y = jnp.dot(h, wo_ref[0], preferred_element_type=jnp.float32)
        y = y.astype(jnp.bfloat16).astype(jnp.float32)
        rows = base + lax.broadcasted_iota(jnp.int32, (TM, 1), 0)
        mask = (rows >= start) & (rows < end)
        cur = out_ref[pl.ds(base, TM), :]
        out_ref[pl.ds(base, TM), :] = jnp.where(mask, y, cur)
        return carry

    lax.fori_loop(0, nchunks, body, 0)


def _gmm(offs65, x_sorted, wi_0, wi_1, wo, interpret=False):
    grid_spec = pltpu.PrefetchScalarGridSpec(
        num_scalar_prefetch=1,
        grid=(E,),
        in_specs=[
            pl.BlockSpec((A, H), lambda e, offs: (0, 0)),
            pl.BlockSpec((1, H, I), lambda e, offs: (e, 0, 0)),
            pl.BlockSpec((1, H, I), lambda e, offs: (e, 0, 0)),
            pl.BlockSpec((1, I, H), lambda e, offs: (e, 0, 0)),
        ],
        out_specs=pl.BlockSpec((A, H), lambda e, offs: (0, 0)),
    )
    return pl.pallas_call(
        _gmm_body,
        grid_spec=grid_spec,
        out_shape=jax.ShapeDtypeStruct((A, H), jnp.float32),
        compiler_params=pltpu.CompilerParams(
            dimension_semantics=("arbitrary",)),
        interpret=interpret,
    )(offs65, x_sorted, wi_0, wi_1, wo)


def _gather_outputs(y, dest_sm):
    @functools.partial(
        pl.kernel,
        out_type=[
            jax.ShapeDtypeStruct((T, H), jnp.float32),
            jax.ShapeDtypeStruct((T, H), jnp.float32),
        ],
        mesh=_sc_mesh(),
        scratch_types=[
            pltpu.VMEM((CHK,), jnp.int32),
            pltpu.VMEM((CHK,), jnp.int32),
            pltpu.VMEM((CHK, H), jnp.float32),
            pltpu.VMEM((CHK, H), jnp.float32),
            pltpu.SemaphoreType.DMA,
        ],
    )
    def k(y_hbm, dest_hbm, y0_hbm, y1_hbm, idx0_v, idx1_v, r0_v, r1_v, sem):
        wid = lax.axis_index("s") * 2 + lax.axis_index("c")
        for half in range(TPW // CHK):
            tok = wid * TPW + half * CHK
            pltpu.sync_copy(dest_hbm.at[0, pl.ds(tok, CHK)], idx0_v)
            pltpu.sync_copy(dest_hbm.at[1, pl.ds(tok, CHK)], idx1_v)
            cp0 = pltpu.async_copy(y_hbm.at[idx0_v], r0_v, sem)
            cp1 = pltpu.async_copy(y_hbm.at[idx1_v], r1_v, sem)
            cp0.wait()
            cp1.wait()
            pltpu.sync_copy(r0_v, y0_hbm.at[pl.ds(tok, CHK)])
            pltpu.sync_copy(r1_v, y1_hbm.at[pl.ds(tok, CHK)])

    return k(y, dest_sm)


def _combine_body(w_ref, y0_ref, y1_ref, out_ref):
    i = pl.program_id(0)
    n = y0_ref.shape[0]
    w = w_ref[pl.ds(i * n, n), :]
    out_ref[...] = y0_ref[...] * w[:, 0:1] + y1_ref[...] * w[:, 1:2]


def _combine(w_cols, y0, y1, interpret=False):
    nblk = 8
    blk = T // nblk
    return pl.pallas_call(
        _combine_body,
        grid=(nblk,),
        in_specs=[
            pl.BlockSpec((T, TK), lambda i: (0, 0)),
            pl.BlockSpec((blk, H), lambda i: (i, 0)),
            pl.BlockSpec((blk, H), lambda i: (i, 0)),
        ],
        out_specs=pl.BlockSpec((blk, H), lambda i: (i, 0)),
        out_shape=jax.ShapeDtypeStruct((T, H), jnp.float32),
        interpret=interpret,
    )(w_cols, y0, y1)


def kernel(inputs, router_logits, wi_0, wi_1, wo):
    dest_cols, w_cols, offs_rows = _routing(router_logits)
    dest_sm = dest_cols.T
    offs65 = jnp.concatenate(
        [offs_rows[0], jnp.array([A], jnp.int32)])
    x_sorted = _dispatch(inputs, dest_sm)
    y = _gmm(offs65, x_sorted, wi_0, wi_1, wo)
    y0, y1 = _gather_outputs(y, dest_sm)
    return _combine(w_cols, y0, y1)
